# baseline (speedup 1.0000x reference)
import functools

import jax
import jax.numpy as jnp
from jax import lax
from jax.experimental import pallas as pl
from jax.experimental.pallas import tpu as pltpu
from jax.experimental.pallas import tpu_sc as plsc

T, M, E = 50, 10, 4096
LANES = 16
ROW = 128
ROWS_OUT = (M * E) // ROW
ROWS_PER_W = LANES
N_WORKERS = ROWS_OUT // ROWS_PER_W
NC, NS = 2, 16

_mesh = plsc.VectorSubcoreMesh(
    core_axis_name="c", subcore_axis_name="s", num_cores=NC, num_subcores=NS
)


@functools.partial(
    pl.kernel,
    out_type=jax.ShapeDtypeStruct((ROWS_OUT, ROW), jnp.float32),
    mesh=_mesh,
    scratch_types=[
        pltpu.VMEM((LANES,), jnp.int32),
        pltpu.VMEM((LANES,), jnp.int32),
        pltpu.VMEM((ROWS_PER_W, ROW), jnp.float32),
        pltpu.SemaphoreType.DMA,
    ],
)
def _lookup(pool_hbm, id_hbm, out_hbm, id_v, idx_v, rows_v, sem):
    wid = lax.axis_index("s") * NC + lax.axis_index("c")

    @pl.when(wid < N_WORKERS)
    def _():
        pltpu.sync_copy(id_hbm, id_v)
        idx_v[...] = (
            id_v[...] * ROWS_OUT
            + wid * ROWS_PER_W
            + lax.iota(jnp.int32, LANES)
        )
        pltpu.async_copy(pool_hbm.at[idx_v], rows_v, sem).wait()
        pltpu.sync_copy(rows_v, out_hbm.at[pl.ds(wid * ROWS_PER_W, ROWS_PER_W)])


def kernel(pool, id):
    pool_rows = pool.reshape(T * ROWS_OUT, ROW)
    id_vec = jnp.full((LANES,), id, dtype=jnp.int32)
    out = _lookup(pool_rows, id_vec)
    return out.reshape(M, E)


# speedup vs baseline: 1.6869x; 1.6869x over previous
import functools

import jax
import jax.numpy as jnp
from jax.experimental import pallas as pl
from jax.experimental.pallas import tpu as pltpu
from jax.experimental.pallas import tpu_sc as plsc

T, M, E = 50, 10, 4096
LANES = 16

_mesh = plsc.ScalarSubcoreMesh(axis_name="c", num_cores=1)


@functools.partial(
    pl.kernel,
    out_type=jax.ShapeDtypeStruct((M, E), jnp.float32),
    mesh=_mesh,
    scratch_types=[
        pltpu.SMEM((LANES,), jnp.int32),
    ],
)
def _lookup(pool_hbm, id_hbm, out_hbm, id_s):
    pltpu.sync_copy(id_hbm, id_s)
    sid = id_s[0]
    pltpu.sync_copy(pool_hbm.at[sid], out_hbm)


def kernel(pool, id):
    id_vec = jnp.full((LANES,), id, dtype=jnp.int32)
    return _lookup(pool, id_vec)
